# Initial kernel scaffold; baseline (speedup 1.0000x reference)
import jax
import jax.numpy as jnp
from jax.experimental import pallas as pl


def kernel(x, mask, weight, bias):
    raise NotImplementedError("write your pallas kernel here")


# speedup vs baseline: 516.0850x; 516.0850x over previous
import functools

import jax
import jax.numpy as jnp
import numpy as np
from jax.experimental import pallas as pl
from jax.experimental.pallas import tpu as pltpu

_K = 3
_NITERS = 30


def _wos_body(u_ref, mp_ref, mm_ref, wp_ref, wm_ref, bias_ref, out_ref):
    c = pl.program_id(1)
    u = u_ref[...]
    mp = mp_ref[0]
    mm = mm_ref[0]
    wp = wp_ref[0]
    wm = wm_ref[0]
    b = bias_ref[c, 0]

    mxp = u + mp
    mxm = mm - u

    hi0 = jnp.maximum(jnp.max(mxp, axis=0, keepdims=True),
                      jnp.max(mxm, axis=0, keepdims=True))
    lo0 = jnp.minimum(jnp.min(mxp, axis=0, keepdims=True),
                      jnp.min(mxm, axis=0, keepdims=True))

    def body(_, carry):
        lo, hi = carry
        t = 0.5 * (lo + hi)
        gp = jnp.sum(jnp.where(mxp >= t, wp, 0.0), axis=0, keepdims=True)
        gm = jnp.sum(jnp.where(mxm >= t, wm, 0.0), axis=0, keepdims=True)
        le = (gp + gm) <= b
        return jnp.where(le, lo, t), jnp.where(le, t, hi)

    lo, hi = jax.lax.fori_loop(0, _NITERS, body, (lo0, hi0))

    inf = jnp.float32(jnp.inf)
    ap = jnp.min(jnp.where(mxp >= hi, mxp, inf), axis=0, keepdims=True)
    am = jnp.min(jnp.where(mxm >= hi, mxm, inf), axis=0, keepdims=True)
    ans = jnp.minimum(ap, am)
    ans = jnp.where(jnp.isfinite(ans), ans, hi0)
    out_ref[...] = ans[None]


def _wos_select(uT, maskp, maskm, wp, wm, bias, n_rows, n_chan, d):
    n_blocks = 7
    rb = n_rows // n_blocks

    return pl.pallas_call(
        _wos_body,
        grid=(n_blocks, n_chan),
        in_specs=[
            pl.BlockSpec((d, rb), lambda r, c: (0, r)),
            pl.BlockSpec((1, d, 1), lambda r, c: (c, 0, 0)),
            pl.BlockSpec((1, d, 1), lambda r, c: (c, 0, 0)),
            pl.BlockSpec((1, d, 1), lambda r, c: (c, 0, 0)),
            pl.BlockSpec((1, d, 1), lambda r, c: (c, 0, 0)),
            pl.BlockSpec(memory_space=pltpu.SMEM),
        ],
        out_specs=pl.BlockSpec((1, 1, rb), lambda r, c: (c, 0, r)),
        out_shape=jax.ShapeDtypeStruct((n_chan, 1, n_rows), jnp.float32),
    )(uT, maskp, maskm, wp, wm, bias)


@jax.jit
def kernel(x, mask, weight, bias):
    b_, c_, h_, w_ = x.shape
    d = c_ * _K * _K
    nc = mask.shape[0]
    l = h_ * w_
    n = b_ * l

    xp = jnp.pad(x, ((0, 0), (0, 0), (1, 1), (1, 1)))
    patches = [xp[:, :, i:i + h_, j:j + w_] for i in range(_K) for j in range(_K)]
    u = jnp.stack(patches, axis=2).reshape(b_, d, l)
    uT = jnp.transpose(u, (1, 0, 2)).reshape(d, n)

    maskp = mask[:, :d, None]
    maskm = mask[:, d:, None]
    wp = weight[:, :d, None]
    wm = weight[:, d:, None]

    y = _wos_select(uT, maskp, maskm, wp, wm, bias, n, nc, d)
    out = y.reshape(nc, b_, l).transpose(1, 0, 2).reshape(b_, nc, h_, w_)
    return out


# speedup vs baseline: 728.9444x; 1.4125x over previous
import functools

import jax
import jax.numpy as jnp
import numpy as np
from jax.experimental import pallas as pl
from jax.experimental.pallas import tpu as pltpu

_K = 3
_NITERS = 20


def _wos_body(u_ref, mp_ref, mm_ref, wp_ref, wm_ref, bias_ref, out_ref):
    c = pl.program_id(1)
    u = u_ref[...]
    mp = mp_ref[0]
    mm = mm_ref[0]
    wp = wp_ref[0]
    wm = wm_ref[0]
    b = bias_ref[c, 0]

    mxp = u + mp
    mxm = mm - u

    hi0 = jnp.maximum(jnp.max(mxp, axis=0, keepdims=True),
                      jnp.max(mxm, axis=0, keepdims=True))
    lo0 = jnp.minimum(jnp.min(mxp, axis=0, keepdims=True),
                      jnp.min(mxm, axis=0, keepdims=True))

    def body(_, carry):
        lo, hi = carry
        t = 0.5 * (lo + hi)
        gp = jnp.sum(jnp.where(mxp >= t, wp, 0.0), axis=0, keepdims=True)
        gm = jnp.sum(jnp.where(mxm >= t, wm, 0.0), axis=0, keepdims=True)
        le = (gp + gm) <= b
        return jnp.where(le, lo, t), jnp.where(le, t, hi)

    lo, hi = jax.lax.fori_loop(0, _NITERS, body, (lo0, hi0))

    inf = jnp.float32(jnp.inf)
    ap = jnp.min(jnp.where(mxp >= hi, mxp, inf), axis=0, keepdims=True)
    am = jnp.min(jnp.where(mxm >= hi, mxm, inf), axis=0, keepdims=True)
    ans = jnp.minimum(ap, am)
    ans = jnp.where(jnp.isfinite(ans), ans, hi0)
    out_ref[...] = ans[None]


def _wos_select(uT, maskp, maskm, wp, wm, bias, n_rows, n_chan, d):
    n_blocks = 7
    rb = n_rows // n_blocks

    return pl.pallas_call(
        _wos_body,
        grid=(n_blocks, n_chan),
        in_specs=[
            pl.BlockSpec((d, rb), lambda r, c: (0, r)),
            pl.BlockSpec((1, d, 1), lambda r, c: (c, 0, 0)),
            pl.BlockSpec((1, d, 1), lambda r, c: (c, 0, 0)),
            pl.BlockSpec((1, d, 1), lambda r, c: (c, 0, 0)),
            pl.BlockSpec((1, d, 1), lambda r, c: (c, 0, 0)),
            pl.BlockSpec(memory_space=pltpu.SMEM),
        ],
        out_specs=pl.BlockSpec((1, 1, rb), lambda r, c: (c, 0, r)),
        out_shape=jax.ShapeDtypeStruct((n_chan, 1, n_rows), jnp.float32),
    )(uT, maskp, maskm, wp, wm, bias)


@jax.jit
def kernel(x, mask, weight, bias):
    b_, c_, h_, w_ = x.shape
    d = c_ * _K * _K
    nc = mask.shape[0]
    l = h_ * w_
    n = b_ * l

    xp = jnp.pad(x, ((0, 0), (0, 0), (1, 1), (1, 1)))
    patches = [xp[:, :, i:i + h_, j:j + w_] for i in range(_K) for j in range(_K)]
    u = jnp.stack(patches, axis=2).reshape(b_, d, l)
    uT = jnp.transpose(u, (1, 0, 2)).reshape(d, n)

    maskp = mask[:, :d, None]
    maskm = mask[:, d:, None]
    wp = weight[:, :d, None]
    wm = weight[:, d:, None]

    y = _wos_select(uT, maskp, maskm, wp, wm, bias, n, nc, d)
    out = y.reshape(nc, b_, l).transpose(1, 0, 2).reshape(b_, nc, h_, w_)
    return out
